# baseline (speedup 1.0000x reference)
import functools

import jax
import jax.numpy as jnp
from jax import lax
from jax.experimental import pallas as pl
from jax.experimental.pallas import tpu as pltpu
from jax.experimental.pallas import tpu_sc as plsc

N = 10000
NP = 10240
E = 320000
D = 128

NC = 2
NS = 16
NW = NC * NS
B = 40
ET = E // NW
NB = ET // B
RPT = NP // NS


def _spmm_body(x_hbm, rows_hbm, cols_hbm, vals_hbm, out_hbm,
               acc, vals_v,
               c0, c1, c2, r0, r1, r2, g0, g1, g2,
               cs0, cs1, cs2, rs0, rs1, rs2,
               gs0, gs1, gs2, ss0, ss1, ss2):
    cbuf = (c0, c1, c2)
    rbuf = (r0, r1, r2)
    gbuf = (g0, g1, g2)
    csem = (cs0, cs1, cs2)
    rsem = (rs0, rs1, rs2)
    gsem = (gs0, gs1, gs2)
    ssem = (ss0, ss1, ss2)

    cid = lax.axis_index("c")
    sid = lax.axis_index("s")
    wid = cid * NS + sid
    ebase = wid * ET

    def zrow(j, c_):
        for c in range(D // 16):
            g0[j, pl.ds(c * 16, 16)] = jnp.zeros((16,), jnp.float32)
        return c_
    lax.fori_loop(0, B, zrow, 0)
    for k in range(RPT // B):
        pltpu.sync_copy(g0, acc.at[pl.ds(sid * RPT + k * B, B)])

    pltpu.sync_copy(vals_hbm.at[pl.ds(ebase, ET)], vals_v.at[pl.ds(0, ET)])

    plsc.subcore_barrier()

    def fetch_cols(t, p):
        pltpu.async_copy(cols_hbm.at[pl.ds(ebase + t * B, B)], cbuf[p], csem[p])

    def fetch_rows(t, p):
        pltpu.async_copy(rows_hbm.at[pl.ds(ebase + t * B, B)], rbuf[p], rsem[p])

    def wait_c(t, p):
        pltpu.make_async_copy(cols_hbm.at[pl.ds(ebase + t * B, B)], cbuf[p],
                              csem[p]).wait()

    def wait_r(t, p):
        pltpu.make_async_copy(rows_hbm.at[pl.ds(ebase + t * B, B)], rbuf[p],
                              rsem[p]).wait()

    def fire_gather(p):
        pltpu.async_copy(x_hbm.at[cbuf[p]], gbuf[p], gsem[p])

    def wait_g(p):
        pltpu.make_async_copy(x_hbm.at[cbuf[p]], gbuf[p], gsem[p]).wait()

    def fire_scatter(p):
        pltpu.async_copy(gbuf[p], acc.at[rbuf[p]], ssem[p], add=True)

    def wait_s(p):
        pltpu.make_async_copy(gbuf[p], acc.at[rbuf[p]], ssem[p]).wait()

    def scale(bi, gb):
        for k in range(B // 8):
            vv = vals_v[pl.ds(bi * B + k * 8, 16)]
            for r in range(8):
                v = vv[r]
                j = k * 8 + r
                for c in range(D // 16):
                    sl = pl.ds(c * 16, 16)
                    gb[j, sl] = gb[j, sl] * v

    def half(t, p, cols2, waits, fire, rows1):
        if cols2:
            fetch_cols(t + 2, (p + 2) % 3)
        if waits:
            wait_s((p + 1) % 3)
        if fire:
            wait_c(t + 1, (p + 1) % 3)
            fire_gather((p + 1) % 3)
        wait_g(p)
        scale(t, gbuf[p])
        if rows1:
            fetch_rows(t + 1, (p + 1) % 3)
        wait_r(t, p)
        fire_scatter(p)

    fetch_cols(0, 0)
    fetch_rows(0, 0)
    fetch_cols(1, 1)
    fetch_rows(1, 1)
    fetch_rows(2, 2)
    wait_c(0, 0)
    fire_gather(0)

    half(0, 0, cols2=True, waits=False, fire=True, rows1=False)
    half(1, 1, cols2=True, waits=False, fire=True, rows1=False)
    half(2, 2, cols2=True, waits=True, fire=True, rows1=True)
    half(3, 0, cols2=True, waits=True, fire=True, rows1=True)

    def body(s, c_):
        t = 3 * s + 1
        half(t, 1, cols2=True, waits=True, fire=True, rows1=True)
        half(t + 1, 2, cols2=True, waits=True, fire=True, rows1=True)
        half(t + 2, 0, cols2=True, waits=True, fire=True, rows1=True)
        return c_
    lax.fori_loop(1, (NB - 4) // 3, body, 0)

    for t in range(3 * ((NB - 4) // 3) + 1, NB - 2):
        half(t, t % 3, cols2=True, waits=True, fire=True, rows1=True)
    half(NB - 2, (NB - 2) % 3, cols2=False, waits=True, fire=True, rows1=True)
    half(NB - 1, (NB - 1) % 3, cols2=False, waits=True, fire=False, rows1=False)

    wait_s((NB - 2) % 3)
    wait_s((NB - 1) % 3)

    plsc.subcore_barrier()

    sl = pl.ds(sid * RPT, RPT)
    pltpu.sync_copy(acc.at[sl], out_hbm.at[cid, sl])


@functools.cache
def _build_spmm():
    return pl.kernel(
        _spmm_body,
        out_type=jax.ShapeDtypeStruct((NC, NP, D), jnp.float32),
        mesh=plsc.VectorSubcoreMesh(
            core_axis_name="c", subcore_axis_name="s",
            num_cores=NC, num_subcores=NS),
        scratch_types=(
            [pltpu.VMEM_SHARED((NP, D), jnp.float32)]
            + [pltpu.VMEM((ET + 16,), jnp.float32)]
            + [pltpu.VMEM((B,), jnp.int32)] * 3
            + [pltpu.VMEM((B,), jnp.int32)] * 3
            + [pltpu.VMEM((B, D), jnp.float32)] * 3
            + [pltpu.SemaphoreType.DMA] * 12
        ),
    )


BM = 1024


def _mm_body(p_ref, w_ref, o_ref):
    x = p_ref[0] + p_ref[1]
    o_ref[...] = lax.dot_general(
        x, w_ref[...], (((1,), (1,)), ((), ())),
        preferred_element_type=jnp.float32)


_mm = pl.pallas_call(
    _mm_body,
    grid=(NP // BM,),
    in_specs=[
        pl.BlockSpec((NC, BM, D), lambda i: (0, i, 0)),
        pl.BlockSpec((D, D), lambda i: (0, 0)),
    ],
    out_specs=pl.BlockSpec((BM, D), lambda i: (i, 0)),
    out_shape=jax.ShapeDtypeStruct((N, D), jnp.float32),
)


def kernel(X, W, rows, cols, vals):
    partials = _build_spmm()(X, rows.astype(jnp.int32), cols.astype(jnp.int32),
                             vals)
    return _mm(partials, W)


# speedup vs baseline: 1.0227x; 1.0227x over previous
import functools

import jax
import jax.numpy as jnp
from jax import lax
from jax.experimental import pallas as pl
from jax.experimental.pallas import tpu as pltpu
from jax.experimental.pallas import tpu_sc as plsc

N = 10000
NP = 10240
E = 320000
D = 128

NC = 2
NS = 16
NW = NC * NS
B = 40
ET = E // NW
NB = ET // B
RPT = NP // NS


def _spmm_body(x_hbm, rows_hbm, cols_hbm, vals_hbm, out_hbm,
               acc, vals_v,
               c0, c1, c2, r0, r1, r2, g0, g1, g2,
               cs0, cs1, cs2, rs0, rs1, rs2,
               gs0, gs1, gs2, ss0, ss1, ss2):
    cbuf = (c0, c1, c2)
    rbuf = (r0, r1, r2)
    gbuf = (g0, g1, g2)
    csem = (cs0, cs1, cs2)
    rsem = (rs0, rs1, rs2)
    gsem = (gs0, gs1, gs2)
    ssem = (ss0, ss1, ss2)

    cid = lax.axis_index("c")
    sid = lax.axis_index("s")
    wid = cid * NS + sid
    ebase = wid * ET

    def zrow(j, c_):
        for c in range(D // 16):
            g0[j, pl.ds(c * 16, 16)] = jnp.zeros((16,), jnp.float32)
        return c_
    lax.fori_loop(0, B, zrow, 0)
    for k in range(RPT // B):
        pltpu.sync_copy(g0, acc.at[pl.ds(sid * RPT + k * B, B)])

    pltpu.sync_copy(vals_hbm.at[pl.ds(ebase, ET)], vals_v.at[pl.ds(0, ET)])

    plsc.subcore_barrier()

    def fetch_cols(t, p):
        pltpu.async_copy(cols_hbm.at[pl.ds(ebase + t * B, B)], cbuf[p], csem[p])

    def fetch_rows(t, p):
        pltpu.async_copy(rows_hbm.at[pl.ds(ebase + t * B, B)], rbuf[p], rsem[p])

    def wait_c(t, p):
        pltpu.make_async_copy(cols_hbm.at[pl.ds(ebase + t * B, B)], cbuf[p],
                              csem[p]).wait()

    def wait_r(t, p):
        pltpu.make_async_copy(rows_hbm.at[pl.ds(ebase + t * B, B)], rbuf[p],
                              rsem[p]).wait()

    def fire_gather(p):
        pltpu.async_copy(x_hbm.at[cbuf[p]], gbuf[p], gsem[p])

    def wait_g(p):
        pltpu.make_async_copy(x_hbm.at[cbuf[p]], gbuf[p], gsem[p]).wait()

    def fire_scatter(p):
        pltpu.async_copy(gbuf[p], acc.at[rbuf[p]], ssem[p], add=True)

    def wait_s(p):
        pltpu.make_async_copy(gbuf[p], acc.at[rbuf[p]], ssem[p]).wait()

    def scale(bi, gb):
        for k in range(B // 8):
            vv = vals_v[pl.ds(bi * B + k * 8, 16)]
            for r in range(8):
                v = vv[r]
                j = k * 8 + r
                for c in range(D // 16):
                    sl = pl.ds(c * 16, 16)
                    gb[j, sl] = gb[j, sl] * v

    def half(t, p, prefetch, fire, first=False):
        if prefetch:
            fetch_cols(t + 2, (p + 2) % 3)
        if fire:
            wait_c(t + 1, (p + 1) % 3)
            fire_gather((p + 1) % 3)
        wait_g(p)
        scale(t, gbuf[p])
        if prefetch:
            if not first:
                wait_s((p + 2) % 3)
            fetch_rows(t + 2, (p + 2) % 3)
        wait_r(t, p)
        fire_scatter(p)

    fetch_cols(0, 0)
    fetch_rows(0, 0)
    fetch_cols(1, 1)
    fetch_rows(1, 1)
    wait_c(0, 0)
    fire_gather(0)

    half(0, 0, prefetch=True, fire=True, first=True)

    def body(s, c_):
        t = 3 * s + 1
        half(t, 1, prefetch=True, fire=True)
        half(t + 1, 2, prefetch=True, fire=True)
        half(t + 2, 0, prefetch=True, fire=True)
        return c_
    lax.fori_loop(0, (NB - 4) // 3, body, 0)

    for t in range(3 * ((NB - 4) // 3) + 1, NB - 2):
        half(t, t % 3, prefetch=True, fire=True)
    half(NB - 2, (NB - 2) % 3, prefetch=False, fire=True)
    half(NB - 1, (NB - 1) % 3, prefetch=False, fire=False)

    wait_s(0)
    wait_s(1)
    wait_s(2)

    plsc.subcore_barrier()

    sl = pl.ds(sid * RPT, RPT)
    pltpu.sync_copy(acc.at[sl], out_hbm.at[cid, sl])


@functools.cache
def _build_spmm():
    return pl.kernel(
        _spmm_body,
        out_type=jax.ShapeDtypeStruct((NC, NP, D), jnp.float32),
        mesh=plsc.VectorSubcoreMesh(
            core_axis_name="c", subcore_axis_name="s",
            num_cores=NC, num_subcores=NS),
        scratch_types=(
            [pltpu.VMEM_SHARED((NP, D), jnp.float32)]
            + [pltpu.VMEM((ET + 16,), jnp.float32)]
            + [pltpu.VMEM((B,), jnp.int32)] * 3
            + [pltpu.VMEM((B,), jnp.int32)] * 3
            + [pltpu.VMEM((B, D), jnp.float32)] * 3
            + [pltpu.SemaphoreType.DMA] * 12
        ),
    )


BM = 1024


def _mm_body(p_ref, w_ref, o_ref):
    x = p_ref[0] + p_ref[1]
    o_ref[...] = lax.dot_general(
        x, w_ref[...], (((1,), (1,)), ((), ())),
        preferred_element_type=jnp.float32)


_mm = pl.pallas_call(
    _mm_body,
    grid=(NP // BM,),
    in_specs=[
        pl.BlockSpec((NC, BM, D), lambda i: (0, i, 0)),
        pl.BlockSpec((D, D), lambda i: (0, 0)),
    ],
    out_specs=pl.BlockSpec((BM, D), lambda i: (i, 0)),
    out_shape=jax.ShapeDtypeStruct((N, D), jnp.float32),
)


def kernel(X, W, rows, cols, vals):
    partials = _build_spmm()(X, rows.astype(jnp.int32), cols.astype(jnp.int32),
                             vals)
    return _mm(partials, W)
